# baseline (speedup 1.0000x reference)
import functools

import jax
import jax.numpy as jnp
from jax import lax
from jax.experimental import pallas as pl
from jax.experimental.pallas import tpu as pltpu
from jax.experimental.pallas import tpu_sc as plsc

VOCAB = 1000000
EMBED_DIM = 32
BATCH = 16384
HIST = 50
N_IDX = BATCH * HIST
NC, NS = 2, 16
NW = NC * NS
PER_TILE = N_IDX // NW
ROWS = 128
N_GATHER = PER_TILE // ROWS
N_PAIR = N_GATHER // 2


SEG = 128 // EMBED_DIM
ROWS_R = VOCAB // SEG
BLK = 2000


def _mv_body(t_ref, w_ref, o_ref):
    o_ref[...] = jnp.dot(t_ref[...], w_ref[...],
                         preferred_element_type=jnp.float32)


_matvec = pl.pallas_call(
    _mv_body,
    grid=(ROWS_R // BLK,),
    in_specs=[
        pl.BlockSpec((BLK, 128), lambda i: (i, 0)),
        pl.BlockSpec((128, SEG), lambda i: (0, 0)),
    ],
    out_specs=pl.BlockSpec((BLK, SEG), lambda i: (i, 0)),
    out_shape=jax.ShapeDtypeStruct((ROWS_R, SEG), jnp.float32),
)


_mesh = plsc.VectorSubcoreMesh(
    core_axis_name="c", subcore_axis_name="s", num_cores=NC, num_subcores=NS
)


@functools.partial(
    pl.kernel,
    out_type=jax.ShapeDtypeStruct((NW, 16), jnp.float32),
    mesh=_mesh,
    compiler_params=pltpu.CompilerParams(use_tc_tiling_on_sc=False),
    scratch_types=[
        pltpu.VMEM((N_GATHER, ROWS), jnp.int32),
        pltpu.VMEM((ROWS,), jnp.float32),
        pltpu.VMEM((ROWS,), jnp.float32),
        pltpu.VMEM((16,), jnp.float32),
        pltpu.VMEM_SHARED((VOCAB,), jnp.float32),
        pltpu.SemaphoreType.DMA,
        pltpu.SemaphoreType.DMA,
    ],
)
def _gather_reduce(idx_hbm, s_hbm, out_hbm,
                   idx_v, buf0, buf1, out_v, s_sh, sem0, sem1):
    wid = lax.axis_index("s") * NC + lax.axis_index("c")
    sid = lax.axis_index("s")
    pltpu.sync_copy(idx_hbm.at[wid], idx_v)

    @pl.when(sid == 0)
    def _stage():
        pltpu.sync_copy(s_hbm, s_sh)

    plsc.subcore_barrier()

    zero = jnp.zeros((16,), jnp.float32)

    def clear(i, _):
        buf0[pl.ds(16 * i, 16)] = zero
        buf1[pl.ds(16 * i, 16)] = zero
        return 0

    lax.fori_loop(0, ROWS // 16, clear, 0)

    def wait(buf, sem):
        pltpu.make_async_copy(s_sh.at[idx_v.at[0]], buf, sem).wait()

    pltpu.async_copy(s_sh.at[idx_v.at[0]], buf0, sem0, add=True)
    pltpu.async_copy(s_sh.at[idx_v.at[1]], buf1, sem1, add=True)

    def pair(t, carry):
        g = 2 * t + 2
        wait(buf0, sem0)
        pltpu.async_copy(s_sh.at[idx_v.at[g]], buf0, sem0, add=True)
        wait(buf1, sem1)
        pltpu.async_copy(s_sh.at[idx_v.at[g + 1]], buf1, sem1, add=True)
        return carry

    lax.fori_loop(0, N_PAIR - 1, pair, 0)
    wait(buf0, sem0)
    wait(buf1, sem1)

    def accum(buf, acc):
        def body(i, a):
            return a + buf[pl.ds(16 * i, 16)]
        return lax.fori_loop(0, ROWS // 16, body, acc, unroll=8)

    acc = accum(buf0, zero)
    acc = accum(buf1, acc)

    out_v[...] = acc
    pltpu.sync_copy(out_v, out_hbm.at[wid])


def kernel(x, table, W, b):
    w = W.reshape(EMBED_DIM)
    lane = jnp.arange(128, dtype=jnp.int32)
    seg = jnp.arange(SEG, dtype=jnp.int32)
    w_exp = jnp.where((lane[:, None] // EMBED_DIM) == seg[None, :],
                      jnp.tile(w, SEG)[:, None], 0.0).astype(jnp.float32)
    s = _matvec(table.reshape(ROWS_R, 128), w_exp).reshape(VOCAB)
    idx = x.reshape(NW, N_GATHER, ROWS)
    partials = _gather_reduce(idx, s)
    return jnp.sum(partials) / jnp.float32(N_IDX) + b[0]


# speedup vs baseline: 4.1305x; 4.1305x over previous
import functools

import jax
import jax.numpy as jnp
from jax import lax
from jax.experimental import pallas as pl
from jax.experimental.pallas import tpu as pltpu
from jax.experimental.pallas import tpu_sc as plsc

VOCAB = 1000000
EMBED_DIM = 32
BATCH = 16384
HIST = 50
N_IDX = BATCH * HIST
NC, NS = 2, 16
NW = NC * NS
PER_TILE = N_IDX // NW
ROWS = 128
N_GATHER = PER_TILE // ROWS
N_PAIR = N_GATHER // 2


BLKL = 8192


def _mv_body(w_ref, t_ref, o_ref):
    o_ref[...] = jnp.dot(w_ref[...], t_ref[...],
                         preferred_element_type=jnp.float32)


_matvec = pl.pallas_call(
    _mv_body,
    grid=(pl.cdiv(VOCAB, BLKL),),
    in_specs=[
        pl.BlockSpec((1, EMBED_DIM), lambda i: (0, 0)),
        pl.BlockSpec((EMBED_DIM, BLKL), lambda i: (0, i)),
    ],
    out_specs=pl.BlockSpec((1, BLKL), lambda i: (0, i)),
    out_shape=jax.ShapeDtypeStruct((1, VOCAB), jnp.float32),
)


_mesh = plsc.VectorSubcoreMesh(
    core_axis_name="c", subcore_axis_name="s", num_cores=NC, num_subcores=NS
)


@functools.partial(
    pl.kernel,
    out_type=jax.ShapeDtypeStruct((NW, 16), jnp.float32),
    mesh=_mesh,
    compiler_params=pltpu.CompilerParams(use_tc_tiling_on_sc=False),
    scratch_types=[
        pltpu.VMEM((PER_TILE,), jnp.int32),
        pltpu.VMEM((ROWS,), jnp.float32),
        pltpu.VMEM((ROWS,), jnp.float32),
        pltpu.VMEM((16,), jnp.float32),
        pltpu.VMEM_SHARED((VOCAB,), jnp.float32),
        pltpu.SemaphoreType.DMA,
        pltpu.SemaphoreType.DMA,
    ],
)
def _gather_reduce(idx_hbm, s_hbm, out_hbm,
                   idx_v, buf0, buf1, out_v, s_sh, sem0, sem1):
    wid = lax.axis_index("s") * NC + lax.axis_index("c")
    sid = lax.axis_index("s")
    pltpu.sync_copy(idx_hbm.at[pl.ds(wid * PER_TILE, PER_TILE)], idx_v)

    @pl.when(sid == 0)
    def _stage():
        pltpu.sync_copy(s_hbm, s_sh)

    plsc.subcore_barrier()

    zero = jnp.zeros((16,), jnp.float32)

    def clear(i, _):
        buf0[pl.ds(16 * i, 16)] = zero
        buf1[pl.ds(16 * i, 16)] = zero
        return 0

    lax.fori_loop(0, ROWS // 16, clear, 0)

    def chunk(g):
        return idx_v.at[pl.ds(g * ROWS, ROWS)]

    def wait(buf, sem):
        pltpu.make_async_copy(s_sh.at[chunk(0)], buf, sem).wait()

    pltpu.async_copy(s_sh.at[chunk(0)], buf0, sem0, add=True)
    pltpu.async_copy(s_sh.at[chunk(1)], buf1, sem1, add=True)

    def pair(t, carry):
        g = 2 * t + 2
        wait(buf0, sem0)
        pltpu.async_copy(s_sh.at[chunk(g)], buf0, sem0, add=True)
        wait(buf1, sem1)
        pltpu.async_copy(s_sh.at[chunk(g + 1)], buf1, sem1, add=True)
        return carry

    lax.fori_loop(0, N_PAIR - 1, pair, 0)
    wait(buf0, sem0)
    wait(buf1, sem1)

    def accum(buf, acc):
        def body(i, a):
            return a + buf[pl.ds(16 * i, 16)]
        return lax.fori_loop(0, ROWS // 16, body, acc, unroll=8)

    acc = accum(buf0, zero)
    acc = accum(buf1, acc)

    out_v[...] = acc
    pltpu.sync_copy(out_v, out_hbm.at[wid])


def kernel(x, table, W, b):
    s = _matvec(W, table.T).reshape(VOCAB)
    idx = x.T.reshape(N_IDX)
    partials = _gather_reduce(idx, s)
    return jnp.sum(partials) / jnp.float32(N_IDX) + b[0]


# speedup vs baseline: 5.6208x; 1.3608x over previous
import functools

import jax
import jax.numpy as jnp
from jax import lax
from jax.experimental import pallas as pl
from jax.experimental.pallas import tpu as pltpu
from jax.experimental.pallas import tpu_sc as plsc

VOCAB = 1000000
EMBED_DIM = 32
BATCH = 16384
HIST = 50
N_IDX = BATCH * HIST
NC, NS = 2, 16
NW = NC * NS
PER_TILE = N_IDX // NW
ROWS = 128
N_GATHER = PER_TILE // ROWS
N_PAIR = N_GATHER // 2


BLKL = 32768


def _mv_body(w_ref, t_ref, o_ref):
    o_ref[...] = jnp.dot(w_ref[...], t_ref[...],
                         preferred_element_type=jnp.float32)


_matvec = pl.pallas_call(
    _mv_body,
    grid=(pl.cdiv(VOCAB, BLKL),),
    in_specs=[
        pl.BlockSpec((1, EMBED_DIM), lambda i: (0, 0)),
        pl.BlockSpec((EMBED_DIM, BLKL), lambda i: (0, i)),
    ],
    out_specs=pl.BlockSpec((1, BLKL), lambda i: (0, i)),
    out_shape=jax.ShapeDtypeStruct((1, VOCAB), jnp.float32),
)


_mesh = plsc.VectorSubcoreMesh(
    core_axis_name="c", subcore_axis_name="s", num_cores=NC, num_subcores=NS
)


@functools.partial(
    pl.kernel,
    out_type=jax.ShapeDtypeStruct((NW, 16), jnp.float32),
    mesh=_mesh,
    compiler_params=pltpu.CompilerParams(use_tc_tiling_on_sc=False),
    scratch_types=[
        pltpu.VMEM((PER_TILE,), jnp.int32),
        pltpu.VMEM((ROWS,), jnp.float32),
        pltpu.VMEM((ROWS,), jnp.float32),
        pltpu.VMEM((16,), jnp.float32),
        pltpu.VMEM_SHARED((VOCAB,), jnp.float32),
        pltpu.SemaphoreType.DMA,
        pltpu.SemaphoreType.DMA,
    ],
)
def _gather_reduce(idx_hbm, s_hbm, out_hbm,
                   idx_v, buf0, buf1, out_v, s_sh, sem0, sem1):
    wid = lax.axis_index("s") * NC + lax.axis_index("c")
    sid = lax.axis_index("s")
    pltpu.sync_copy(idx_hbm.at[pl.ds(wid * PER_TILE, PER_TILE)], idx_v)

    @pl.when(sid == 0)
    def _stage():
        pltpu.sync_copy(s_hbm, s_sh)

    plsc.subcore_barrier()

    zero = jnp.zeros((16,), jnp.float32)

    def clear(i, _):
        buf0[pl.ds(16 * i, 16)] = zero
        buf1[pl.ds(16 * i, 16)] = zero
        return 0

    lax.fori_loop(0, ROWS // 16, clear, 0)

    def chunk(g):
        return idx_v.at[pl.ds(g * ROWS, ROWS)]

    def wait(buf, sem):
        pltpu.make_async_copy(s_sh.at[chunk(0)], buf, sem).wait()

    pltpu.async_copy(s_sh.at[chunk(0)], buf0, sem0, add=True)
    pltpu.async_copy(s_sh.at[chunk(1)], buf1, sem1, add=True)

    def pair(t, carry):
        g = 2 * t + 2
        wait(buf0, sem0)
        pltpu.async_copy(s_sh.at[chunk(g)], buf0, sem0, add=True)
        wait(buf1, sem1)
        pltpu.async_copy(s_sh.at[chunk(g + 1)], buf1, sem1, add=True)
        return carry

    lax.fori_loop(0, N_PAIR - 1, pair, 0)
    wait(buf0, sem0)
    wait(buf1, sem1)

    def accum(buf, acc):
        def body(i, a):
            return a + buf[pl.ds(16 * i, 16)]
        return lax.fori_loop(0, ROWS // 16, body, acc, unroll=8)

    acc = accum(buf0, zero)
    acc = accum(buf1, acc)

    out_v[...] = acc
    pltpu.sync_copy(out_v, out_hbm.at[wid])


def kernel(x, table, W, b):
    s = _matvec(W, table.T).reshape(VOCAB)
    idx = x.T.reshape(N_IDX)
    partials = _gather_reduce(idx, s)
    return jnp.sum(partials) / jnp.float32(N_IDX) + b[0]


# speedup vs baseline: 8.1676x; 1.4531x over previous
import functools

import jax
import jax.numpy as jnp
from jax import lax
from jax.experimental import pallas as pl
from jax.experimental.pallas import tpu as pltpu
from jax.experimental.pallas import tpu_sc as plsc

VOCAB = 1000000
EMBED_DIM = 32
BATCH = 16384
HIST = 50
N_IDX = BATCH * HIST
NC, NS = 2, 16
NW = NC * NS
PER_TILE = N_IDX // NW
ROWS = 128
N_GATHER = PER_TILE // ROWS
N_PAIR = N_GATHER // 2
N_STAGE = 8
STAGE_SZ = VOCAB // N_STAGE


BLKL = 32768


def _mv_body(w_ref, t_ref, o_ref):
    o_ref[...] = jnp.dot(w_ref[...], t_ref[...],
                         preferred_element_type=jnp.float32)[0]


_matvec = pl.pallas_call(
    _mv_body,
    grid=(pl.cdiv(VOCAB, BLKL),),
    in_specs=[
        pl.BlockSpec((1, EMBED_DIM), lambda i: (0, 0)),
        pl.BlockSpec((EMBED_DIM, BLKL), lambda i: (0, i)),
    ],
    out_specs=pl.BlockSpec((BLKL,), lambda i: (i,)),
    out_shape=jax.ShapeDtypeStruct((VOCAB,), jnp.float32),
)


_mesh = plsc.VectorSubcoreMesh(
    core_axis_name="c", subcore_axis_name="s", num_cores=NC, num_subcores=NS
)


@functools.partial(
    pl.kernel,
    out_type=jax.ShapeDtypeStruct((NW, 16), jnp.float32),
    mesh=_mesh,
    compiler_params=pltpu.CompilerParams(use_tc_tiling_on_sc=False),
    scratch_types=[
        pltpu.VMEM((PER_TILE,), jnp.int32),
        pltpu.VMEM((ROWS,), jnp.float32),
        pltpu.VMEM((ROWS,), jnp.float32),
        pltpu.VMEM((16,), jnp.float32),
        pltpu.VMEM_SHARED((VOCAB,), jnp.float32),
        pltpu.SemaphoreType.DMA,
        pltpu.SemaphoreType.DMA,
    ],
)
def _gather_reduce(idx_hbm, s_hbm, out_hbm,
                   idx_v, buf0, buf1, out_v, s_sh, sem0, sem1):
    wid = lax.axis_index("s") * NC + lax.axis_index("c")
    sid = lax.axis_index("s")
    pltpu.sync_copy(idx_hbm.at[pl.ds(wid * PER_TILE, PER_TILE)], idx_v)

    @pl.when(sid < N_STAGE)
    def _stage():
        off = sid * STAGE_SZ
        pltpu.sync_copy(s_hbm.at[pl.ds(off, STAGE_SZ)],
                        s_sh.at[pl.ds(off, STAGE_SZ)])

    plsc.subcore_barrier()

    zero = jnp.zeros((16,), jnp.float32)

    def clear(i, _):
        buf0[pl.ds(16 * i, 16)] = zero
        buf1[pl.ds(16 * i, 16)] = zero
        return 0

    lax.fori_loop(0, ROWS // 16, clear, 0)

    def chunk(g):
        return idx_v.at[pl.ds(g * ROWS, ROWS)]

    def wait(buf, sem):
        pltpu.make_async_copy(s_sh.at[chunk(0)], buf, sem).wait()

    pltpu.async_copy(s_sh.at[chunk(0)], buf0, sem0, add=True)
    pltpu.async_copy(s_sh.at[chunk(1)], buf1, sem1, add=True)

    def pair(t, carry):
        g = 2 * t + 2
        wait(buf0, sem0)
        pltpu.async_copy(s_sh.at[chunk(g)], buf0, sem0, add=True)
        wait(buf1, sem1)
        pltpu.async_copy(s_sh.at[chunk(g + 1)], buf1, sem1, add=True)
        return carry

    lax.fori_loop(0, N_PAIR - 1, pair, 0)
    wait(buf0, sem0)
    wait(buf1, sem1)

    def accum(buf, acc):
        def body(i, a):
            return a + buf[pl.ds(16 * i, 16)]
        return lax.fori_loop(0, ROWS // 16, body, acc, unroll=8)

    acc = accum(buf0, zero)
    acc = accum(buf1, acc)

    out_v[...] = acc
    pltpu.sync_copy(out_v, out_hbm.at[wid])


def kernel(x, table, W, b):
    s = _matvec(W, table.T)
    idx = x.T.reshape(N_IDX)
    partials = _gather_reduce(idx, s)
    return jnp.sum(partials) / jnp.float32(N_IDX) + b[0]


# speedup vs baseline: 8.5690x; 1.0491x over previous
import functools

import jax
import jax.numpy as jnp
from jax import lax
from jax.experimental import pallas as pl
from jax.experimental.pallas import tpu as pltpu
from jax.experimental.pallas import tpu_sc as plsc

VOCAB = 1000000
EMBED_DIM = 32
BATCH = 16384
HIST = 50
N_IDX = BATCH * HIST
NC, NS = 2, 16
NW = NC * NS
PER_TILE = N_IDX // NW
ROWS = 128
N_GATHER = PER_TILE // ROWS
N_PAIR = N_GATHER // 2
N_STAGE = 8
STAGE_SZ = VOCAB // N_STAGE


BLKL = 65536


def _mv_body(w_ref, t_ref, o_ref):
    o_ref[...] = jnp.dot(w_ref[...], t_ref[...],
                         preferred_element_type=jnp.float32)[0]


_matvec = pl.pallas_call(
    _mv_body,
    grid=(pl.cdiv(VOCAB, BLKL),),
    in_specs=[
        pl.BlockSpec((1, EMBED_DIM), lambda i: (0, 0)),
        pl.BlockSpec((EMBED_DIM, BLKL), lambda i: (0, i)),
    ],
    out_specs=pl.BlockSpec((BLKL,), lambda i: (i,)),
    out_shape=jax.ShapeDtypeStruct((VOCAB,), jnp.float32),
)


_mesh = plsc.VectorSubcoreMesh(
    core_axis_name="c", subcore_axis_name="s", num_cores=NC, num_subcores=NS
)


@functools.partial(
    pl.kernel,
    out_type=jax.ShapeDtypeStruct((NW, 16), jnp.float32),
    mesh=_mesh,
    compiler_params=pltpu.CompilerParams(use_tc_tiling_on_sc=False),
    scratch_types=[
        pltpu.VMEM((PER_TILE,), jnp.int32),
        pltpu.VMEM((ROWS,), jnp.float32),
        pltpu.VMEM((ROWS,), jnp.float32),
        pltpu.VMEM((16,), jnp.float32),
        pltpu.VMEM_SHARED((VOCAB,), jnp.float32),
        pltpu.SemaphoreType.DMA,
        pltpu.SemaphoreType.DMA,
    ],
)
def _gather_reduce(idx_hbm, s_hbm, out_hbm,
                   idx_v, buf0, buf1, out_v, s_sh, sem0, sem1):
    wid = lax.axis_index("s") * NC + lax.axis_index("c")
    sid = lax.axis_index("s")
    pltpu.sync_copy(idx_hbm.at[pl.ds(wid * PER_TILE, PER_TILE)], idx_v)

    @pl.when(sid < N_STAGE)
    def _stage():
        off = sid * STAGE_SZ
        pltpu.sync_copy(s_hbm.at[pl.ds(off, STAGE_SZ)],
                        s_sh.at[pl.ds(off, STAGE_SZ)])

    plsc.subcore_barrier()

    zero = jnp.zeros((16,), jnp.float32)

    def clear(i, _):
        buf0[pl.ds(16 * i, 16)] = zero
        buf1[pl.ds(16 * i, 16)] = zero
        return 0

    lax.fori_loop(0, ROWS // 16, clear, 0)

    def chunk(g):
        return idx_v.at[pl.ds(g * ROWS, ROWS)]

    def wait(buf, sem):
        pltpu.make_async_copy(s_sh.at[chunk(0)], buf, sem).wait()

    pltpu.async_copy(s_sh.at[chunk(0)], buf0, sem0, add=True)
    pltpu.async_copy(s_sh.at[chunk(1)], buf1, sem1, add=True)

    def pair(t, carry):
        g = 2 * t + 2
        wait(buf0, sem0)
        pltpu.async_copy(s_sh.at[chunk(g)], buf0, sem0, add=True)
        wait(buf1, sem1)
        pltpu.async_copy(s_sh.at[chunk(g + 1)], buf1, sem1, add=True)
        return carry

    lax.fori_loop(0, N_PAIR - 1, pair, 0)
    wait(buf0, sem0)
    wait(buf1, sem1)

    def accum(buf, acc):
        def body(i, a):
            return a + buf[pl.ds(16 * i, 16)]
        return lax.fori_loop(0, ROWS // 16, body, acc, unroll=8)

    acc = accum(buf0, zero)
    acc = accum(buf1, acc)

    out_v[...] = acc
    pltpu.sync_copy(out_v, out_hbm.at[wid])


def kernel(x, table, W, b):
    s = _matvec(W, table.T)
    idx = x.T.reshape(N_IDX)
    partials = _gather_reduce(idx, s)
    return jnp.sum(partials) / jnp.float32(N_IDX) + b[0]
